# baseline (speedup 1.0000x reference)
import functools

import jax
import jax.numpy as jnp
from jax import lax
from jax.experimental import pallas as pl
from jax.experimental.pallas import tpu as pltpu
from jax.experimental.pallas import tpu_sc as plsc

_D = 64
_B = 16384
_R = 3 * _B
_NC, _NS, _L = 2, 16, 16
_NW = _NC * _NS
_RPW = _R // _NW
_C = 64
_NCH = _RPW // _C


@functools.partial(
    pl.kernel,
    mesh=plsc.VectorSubcoreMesh(core_axis_name="c", subcore_axis_name="s"),
    out_type=jax.ShapeDtypeStruct((_R // 2, 2 * _D), jnp.float32),
    compiler_params=pltpu.CompilerParams(use_tc_tiling_on_sc=False),
    scratch_types=[
        pltpu.VMEM((_C,), jnp.int32),
        pltpu.VMEM((9, _C), jnp.int32),
        pltpu.VMEM((9, _C + _L), jnp.int32),
        pltpu.VMEM((_C + _L,), jnp.int32),
        pltpu.VMEM((9, _C, 2 * _D), jnp.float32),
        pltpu.VMEM((_C // 2, 2 * _D), jnp.float32),
        pltpu.VMEM((3 * _D,), jnp.float32),
        pltpu.SemaphoreType.DMA,
    ],
)
def _sc_encode(xc0, xc1, xc2, xc3, xc4, xc5, xc6, xc7, xc8, xc9,
               idx_all, e0, e1, e2, e3, e4, e5, e6, e7, e8, e9,
               hout, idx_v, cols, offs, tlist, rbuf, hloc, emb0v, sem):
    embs = (e1, e2, e3, e4, e5, e6, e7, e8, e9)
    xcs = (xc1, xc2, xc3, xc4, xc5, xc6, xc7, xc8, xc9)
    wid = lax.axis_index("s") * _NC + lax.axis_index("c")
    pltpu.sync_copy(e0, emb0v)
    e0sl = [[emb0v[pl.ds(t * _D + dv * _L, _L)] for dv in range(_D // _L)]
            for t in range(3)]
    base_w = wid * _RPW

    def chunk(ci, carry):
        base = base_w + ci * _C
        pltpu.sync_copy(idx_all.at[pl.ds(base, _C)], idx_v)
        xcps = [pltpu.async_copy(xc0.at[idx_v], tlist.at[pl.ds(0, _C)], sem)]
        xcps += [pltpu.async_copy(xcs[f].at[idx_v], cols.at[f, pl.ds(0, _C)], sem)
                 for f in range(9)]
        for cp in xcps:
            cp.wait()
        for g in range(_C // _L):
            sl = pl.ds(g * _L, _L)
            for f in range(9):
                cv = cols[f, sl]
                offs[f, sl] = (cv & 1) * _D
                cols[f, sl] = cv >> 1
        cps = [pltpu.async_copy(embs[f].at[cols.at[f, pl.ds(0, _C)]],
                                rbuf.at[f], sem)
               for f in range(9)]
        for cp in cps:
            cp.wait()

        def row(r, c2):
            t = tlist[pl.ds(r, _L)][0]
            s0 = jnp.where(t == 0, 1.0, 0.0)
            s1 = jnp.where(t == 1, 1.0, 0.0)
            s2 = jnp.where(t == 2, 1.0, 0.0)
            b0 = lax.broadcast(s0, (_L,))
            b1 = lax.broadcast(s1, (_L,))
            b2 = lax.broadcast(s2, (_L,))
            o = [offs[f, pl.ds(r, _L)][0] for f in range(9)]
            hrow = r >> 1
            hcol = (r & 1) * _D
            for dv in range(_D // _L):
                d0 = dv * _L
                h0 = (e0sl[0][dv] * b0 + e0sl[1][dv] * b1 + e0sl[2][dv] * b2)
                a = (rbuf[0, r, pl.ds(o[0] + d0, _L)]
                     + rbuf[1, r, pl.ds(o[1] + d0, _L)]
                     + rbuf[2, r, pl.ds(o[2] + d0, _L)]
                     + rbuf[3, r, pl.ds(o[3] + d0, _L)])
                b = rbuf[4, r, pl.ds(o[4] + d0, _L)]
                c = (rbuf[5, r, pl.ds(o[5] + d0, _L)]
                     + rbuf[6, r, pl.ds(o[6] + d0, _L)]
                     + rbuf[7, r, pl.ds(o[7] + d0, _L)]
                     + rbuf[8, r, pl.ds(o[8] + d0, _L)])
                hloc[hrow, pl.ds(hcol + d0, _L)] = h0 + a * b0 + b * b1 + c * b2
            return c2

        lax.fori_loop(0, _C, row, 0)
        pltpu.sync_copy(hloc, hout.at[pl.ds(base // 2, _C // 2)])
        return carry

    lax.fori_loop(0, _NCH, chunk, 0)


_BLK2 = 1024
_NB2 = (_B // 2) // _BLK2


def _tc_epilogue(h2, bm2, ew, eb, ow, ob):
    def body(hs, hp, hn, bmr, ewr, ebr, owr, obr, opos, oneg):
        w = owr[...]
        u = jnp.sum(w.T * ewr[...], axis=0, keepdims=True)
        const = jnp.sum(ebr[...] * w[0]) + obr[...][0]
        u2 = jnp.concatenate([u, u], axis=1)
        mlo_m = (jnp.arange(54)[None, :] < 27).astype(jnp.float32)
        bmv = bmr[...]
        ms_lo = jnp.sum(bmv * u2 * mlo_m, axis=1, keepdims=True) + const
        ms_hi = jnp.sum(bmv * u2 * (1.0 - mlo_m), axis=1, keepdims=True) + const
        w2 = jnp.concatenate([w, w], axis=1)
        dlo = (jnp.arange(2 * _D)[None, :] < _D).astype(jnp.float32)
        pos = jnp.maximum(hs[...] + hp[...], 0.0) * w2
        neg = jnp.maximum(hs[...] + hn[...], 0.0) * w2
        opos[...] = jnp.concatenate(
            [jnp.sum(pos * dlo, axis=1, keepdims=True) + ms_lo,
             jnp.sum(pos * (1.0 - dlo), axis=1, keepdims=True) + ms_hi], axis=1)
        oneg[...] = jnp.concatenate(
            [jnp.sum(neg * dlo, axis=1, keepdims=True) + ms_lo,
             jnp.sum(neg * (1.0 - dlo), axis=1, keepdims=True) + ms_hi], axis=1)

    return pl.pallas_call(
        body,
        grid=(_NB2,),
        in_specs=[
            pl.BlockSpec((_BLK2, 2 * _D), lambda i: (i, 0)),
            pl.BlockSpec((_BLK2, 2 * _D), lambda i: (i + _NB2, 0)),
            pl.BlockSpec((_BLK2, 2 * _D), lambda i: (i + 2 * _NB2, 0)),
            pl.BlockSpec((_BLK2, 54), lambda i: (i, 0)),
            pl.BlockSpec((_D, 27), lambda i: (0, 0)),
            pl.BlockSpec((_D,), lambda i: (0,)),
            pl.BlockSpec((1, _D), lambda i: (0, 0)),
            pl.BlockSpec((1,), lambda i: (0,)),
        ],
        out_specs=[
            pl.BlockSpec((_BLK2, 2), lambda i: (i, 0)),
            pl.BlockSpec((_BLK2, 2), lambda i: (i, 0)),
        ],
        out_shape=[
            jax.ShapeDtypeStruct((_B // 2, 2), jnp.float32),
            jax.ShapeDtypeStruct((_B // 2, 2), jnp.float32),
        ],
    )(h2, h2, h2, bm2, ew, eb, ow, ob)


def kernel(x, src, dst, neg_dst, batch_msg,
           emb0, emb1, emb2, emb3, emb4, emb5, emb6, emb7, emb8, emb9,
           edge_W, edge_b, out_W, out_b):
    xt = x.T
    xcs = [xt[f] for f in range(10)]
    idx_all = jnp.concatenate([src, dst, neg_dst], axis=0)
    embs2 = [e.reshape(-1, 2 * _D)
             for e in (emb1, emb2, emb3, emb4, emb5, emb6, emb7, emb8, emb9)]
    h2 = _sc_encode(*xcs, idx_all, emb0.reshape(-1), *embs2)
    bm2 = batch_msg.reshape(_B // 2, 54)
    op2, on2 = _tc_epilogue(h2, bm2, edge_W, edge_b, out_W, out_b)
    return (op2.reshape(_B, 1), on2.reshape(_B, 1))


# speedup vs baseline: 1.0710x; 1.0710x over previous
import functools

import jax
import jax.numpy as jnp
from jax import lax
from jax.experimental import pallas as pl
from jax.experimental.pallas import tpu as pltpu
from jax.experimental.pallas import tpu_sc as plsc

_V = 100000
_D = 64
_B = 16384
_R = 3 * _B
_NC, _NS, _L = 2, 16, 16
_NW = _NC * _NS
_RPW = _R // _NW
_C = 64
_NCH = _RPW // _C

_TBLK = 10000


def _tc_transpose_table(et):
    def body(inr, outr):
        outr[:, 0:_D] = jnp.transpose(inr[...])

    return pl.pallas_call(
        body,
        grid=(_V // _TBLK,),
        in_specs=[pl.BlockSpec((_D, _TBLK), lambda i: (0, i))],
        out_specs=pl.BlockSpec((_TBLK, 2 * _D), lambda i: (i, 0)),
        out_shape=jax.ShapeDtypeStruct((_V, 2 * _D), jnp.float32),
    )(et)


@functools.partial(
    pl.kernel,
    mesh=plsc.VectorSubcoreMesh(core_axis_name="c", subcore_axis_name="s"),
    out_type=jax.ShapeDtypeStruct((_R // 2, 2 * _D), jnp.float32),
    compiler_params=pltpu.CompilerParams(use_tc_tiling_on_sc=False),
    scratch_types=[
        pltpu.VMEM((_C,), jnp.int32),
        pltpu.VMEM((9, _C), jnp.int32),
        pltpu.VMEM((_C + _L,), jnp.int32),
        pltpu.VMEM((9, _C, 2 * _D), jnp.float32),
        pltpu.VMEM((_C // 2, 2 * _D), jnp.float32),
        pltpu.VMEM((3 * _D,), jnp.float32),
        pltpu.SemaphoreType.DMA,
    ],
)
def _sc_encode(xc0, xc1, xc2, xc3, xc4, xc5, xc6, xc7, xc8, xc9,
               idx_all, e0, e1, e2, e3, e4, e5, e6, e7, e8, e9,
               hout, idx_v, cols, tlist, rbuf, hloc, emb0v, sem):
    embs = (e1, e2, e3, e4, e5, e6, e7, e8, e9)
    xcs = (xc1, xc2, xc3, xc4, xc5, xc6, xc7, xc8, xc9)
    wid = lax.axis_index("s") * _NC + lax.axis_index("c")
    pltpu.sync_copy(e0, emb0v)
    e0sl = [[emb0v[pl.ds(t * _D + dv * _L, _L)] for dv in range(_D // _L)]
            for t in range(3)]
    base_w = wid * _RPW

    def chunk(ci, carry):
        base = base_w + ci * _C
        pltpu.sync_copy(idx_all.at[pl.ds(base, _C)], idx_v)
        xcps = [pltpu.async_copy(xc0.at[idx_v], tlist.at[pl.ds(0, _C)], sem)]
        xcps += [pltpu.async_copy(xcs[f].at[idx_v], cols.at[f], sem)
                 for f in range(9)]
        for cp in xcps:
            cp.wait()
        cps = [pltpu.async_copy(embs[f].at[cols.at[f]], rbuf.at[f], sem)
               for f in range(9)]
        for cp in cps:
            cp.wait()

        def row(r, c2):
            t = tlist[pl.ds(r, _L)][0]
            s0 = jnp.where(t == 0, 1.0, 0.0)
            s1 = jnp.where(t == 1, 1.0, 0.0)
            s2 = jnp.where(t == 2, 1.0, 0.0)
            b0 = lax.broadcast(s0, (_L,))
            b1 = lax.broadcast(s1, (_L,))
            b2 = lax.broadcast(s2, (_L,))
            hrow = r >> 1
            hcol = (r & 1) * _D
            for dv in range(_D // _L):
                sl = pl.ds(dv * _L, _L)
                h0 = (e0sl[0][dv] * b0 + e0sl[1][dv] * b1 + e0sl[2][dv] * b2)
                a = (rbuf[0, r, sl] + rbuf[1, r, sl]
                     + rbuf[2, r, sl] + rbuf[3, r, sl])
                b = rbuf[4, r, sl]
                c = (rbuf[5, r, sl] + rbuf[6, r, sl]
                     + rbuf[7, r, sl] + rbuf[8, r, sl])
                hloc[hrow, pl.ds(hcol + dv * _L, _L)] = (
                    h0 + a * b0 + b * b1 + c * b2)
            return c2

        lax.fori_loop(0, _C, row, 0)
        pltpu.sync_copy(hloc, hout.at[pl.ds(base // 2, _C // 2)])
        return carry

    lax.fori_loop(0, _NCH, chunk, 0)


_BLK2 = 1024
_NB2 = (_B // 2) // _BLK2


def _tc_epilogue(h2, bm2, ew, eb, ow, ob):
    def body(hs, hp, hn, bmr, ewr, ebr, owr, obr, opos, oneg):
        w = owr[...]
        u = jnp.sum(w.T * ewr[...], axis=0, keepdims=True)
        const = jnp.sum(ebr[...] * w[0]) + obr[...][0]
        u2 = jnp.concatenate([u, u], axis=1)
        mlo_m = (jnp.arange(54)[None, :] < 27).astype(jnp.float32)
        bmv = bmr[...]
        ms_lo = jnp.sum(bmv * u2 * mlo_m, axis=1, keepdims=True) + const
        ms_hi = jnp.sum(bmv * u2 * (1.0 - mlo_m), axis=1, keepdims=True) + const
        w2 = jnp.concatenate([w, w], axis=1)
        dlo = (jnp.arange(2 * _D)[None, :] < _D).astype(jnp.float32)
        pos = jnp.maximum(hs[...] + hp[...], 0.0) * w2
        neg = jnp.maximum(hs[...] + hn[...], 0.0) * w2
        opos[...] = jnp.concatenate(
            [jnp.sum(pos * dlo, axis=1, keepdims=True) + ms_lo,
             jnp.sum(pos * (1.0 - dlo), axis=1, keepdims=True) + ms_hi], axis=1)
        oneg[...] = jnp.concatenate(
            [jnp.sum(neg * dlo, axis=1, keepdims=True) + ms_lo,
             jnp.sum(neg * (1.0 - dlo), axis=1, keepdims=True) + ms_hi], axis=1)

    return pl.pallas_call(
        body,
        grid=(_NB2,),
        in_specs=[
            pl.BlockSpec((_BLK2, 2 * _D), lambda i: (i, 0)),
            pl.BlockSpec((_BLK2, 2 * _D), lambda i: (i + _NB2, 0)),
            pl.BlockSpec((_BLK2, 2 * _D), lambda i: (i + 2 * _NB2, 0)),
            pl.BlockSpec((_BLK2, 54), lambda i: (i, 0)),
            pl.BlockSpec((_D, 27), lambda i: (0, 0)),
            pl.BlockSpec((_D,), lambda i: (0,)),
            pl.BlockSpec((1, _D), lambda i: (0, 0)),
            pl.BlockSpec((1,), lambda i: (0,)),
        ],
        out_specs=[
            pl.BlockSpec((_BLK2, 2), lambda i: (i, 0)),
            pl.BlockSpec((_BLK2, 2), lambda i: (i, 0)),
        ],
        out_shape=[
            jax.ShapeDtypeStruct((_B // 2, 2), jnp.float32),
            jax.ShapeDtypeStruct((_B // 2, 2), jnp.float32),
        ],
    )(h2, h2, h2, bm2, ew, eb, ow, ob)


def kernel(x, src, dst, neg_dst, batch_msg,
           emb0, emb1, emb2, emb3, emb4, emb5, emb6, emb7, emb8, emb9,
           edge_W, edge_b, out_W, out_b):
    xt = x.T
    xcs = [xt[f] for f in range(10)]
    idx_all = jnp.concatenate([src, dst, neg_dst], axis=0)
    embs2 = [jnp.pad(e, ((0, 0), (0, _D)))
             for e in (emb1, emb2, emb3, emb4, emb5, emb6, emb7, emb8, emb9)]
    h2 = _sc_encode(*xcs, idx_all, emb0.reshape(-1), *embs2)
    bm2 = batch_msg.reshape(_B // 2, 54)
    op2, on2 = _tc_epilogue(h2, bm2, edge_W, edge_b, out_W, out_b)
    return (op2.reshape(_B, 1), on2.reshape(_B, 1))


# speedup vs baseline: 1.1124x; 1.0386x over previous
import functools

import jax
import jax.numpy as jnp
from jax import lax
from jax.experimental import pallas as pl
from jax.experimental.pallas import tpu as pltpu
from jax.experimental.pallas import tpu_sc as plsc

_V = 100000
_D = 64
_B = 16384
_R = 3 * _B
_NC, _NS, _L = 2, 16, 16
_NW = _NC * _NS
_RPW = _R // _NW
_C = 64
_NCH = _RPW // _C

_VP = 102400
_TBLK = 12800


def _tc_transpose_table(etp):
    def body(inr, outr):
        outr[:, 0:_D] = jnp.transpose(inr[...])

    return pl.pallas_call(
        body,
        grid=(_VP // _TBLK,),
        in_specs=[pl.BlockSpec((_D, _TBLK), lambda i: (0, i))],
        out_specs=pl.BlockSpec((_TBLK, 2 * _D), lambda i: (i, 0)),
        out_shape=jax.ShapeDtypeStruct((_VP, 2 * _D), jnp.float32),
    )(etp)


@functools.partial(
    pl.kernel,
    mesh=plsc.VectorSubcoreMesh(core_axis_name="c", subcore_axis_name="s"),
    out_type=jax.ShapeDtypeStruct((_R // 2, 2 * _D), jnp.float32),
    compiler_params=pltpu.CompilerParams(use_tc_tiling_on_sc=False),
    scratch_types=[
        pltpu.VMEM((_C,), jnp.int32),
        pltpu.VMEM((9, _C), jnp.int32),
        pltpu.VMEM((_C + _L,), jnp.int32),
        pltpu.VMEM((9, _C, 2 * _D), jnp.float32),
        pltpu.VMEM((_C // 2, 2 * _D), jnp.float32),
        pltpu.VMEM((3 * _D,), jnp.float32),
        pltpu.SemaphoreType.DMA,
    ],
)
def _sc_encode(xc0, xc1, xc2, xc3, xc4, xc5, xc6, xc7, xc8, xc9,
               idx_all, e0, e1, e2, e3, e4, e5, e6, e7, e8, e9,
               hout, idx_v, cols, tlist, rbuf, hloc, emb0v, sem):
    embs = (e1, e2, e3, e4, e5, e6, e7, e8, e9)
    xcs = (xc1, xc2, xc3, xc4, xc5, xc6, xc7, xc8, xc9)
    wid = lax.axis_index("s") * _NC + lax.axis_index("c")
    pltpu.sync_copy(e0, emb0v)
    e0sl = [[emb0v[pl.ds(t * _D + dv * _L, _L)] for dv in range(_D // _L)]
            for t in range(3)]
    base_w = wid * _RPW

    def chunk(ci, carry):
        base = base_w + ci * _C
        pltpu.sync_copy(idx_all.at[pl.ds(base, _C)], idx_v)
        xcps = [pltpu.async_copy(xc0.at[idx_v], tlist.at[pl.ds(0, _C)], sem)]
        xcps += [pltpu.async_copy(xcs[f].at[idx_v], cols.at[f], sem)
                 for f in range(9)]
        for cp in xcps:
            cp.wait()
        cps = [pltpu.async_copy(embs[f].at[cols.at[f]], rbuf.at[f], sem)
               for f in range(9)]
        for cp in cps:
            cp.wait()

        def row(r, c2):
            t = tlist[pl.ds(r, _L)][0]
            s0 = jnp.where(t == 0, 1.0, 0.0)
            s1 = jnp.where(t == 1, 1.0, 0.0)
            s2 = jnp.where(t == 2, 1.0, 0.0)
            b0 = lax.broadcast(s0, (_L,))
            b1 = lax.broadcast(s1, (_L,))
            b2 = lax.broadcast(s2, (_L,))
            hrow = r >> 1
            hcol = (r & 1) * _D
            for dv in range(_D // _L):
                sl = pl.ds(dv * _L, _L)
                h0 = (e0sl[0][dv] * b0 + e0sl[1][dv] * b1 + e0sl[2][dv] * b2)
                a = (rbuf[0, r, sl] + rbuf[1, r, sl]
                     + rbuf[2, r, sl] + rbuf[3, r, sl])
                b = rbuf[4, r, sl]
                c = (rbuf[5, r, sl] + rbuf[6, r, sl]
                     + rbuf[7, r, sl] + rbuf[8, r, sl])
                hloc[hrow, pl.ds(hcol + dv * _L, _L)] = (
                    h0 + a * b0 + b * b1 + c * b2)
            return c2

        lax.fori_loop(0, _C, row, 0)
        pltpu.sync_copy(hloc, hout.at[pl.ds(base // 2, _C // 2)])
        return carry

    lax.fori_loop(0, _NCH, chunk, 0)


_BLK2 = 1024
_NB2 = (_B // 2) // _BLK2


def _tc_epilogue(h2, bm2, ew, eb, ow, ob):
    def body(hs, hp, hn, bmr, ewr, ebr, owr, obr, opos, oneg):
        w = owr[...]
        u = jnp.sum(w.T * ewr[...], axis=0, keepdims=True)
        const = jnp.sum(ebr[...] * w[0]) + obr[...][0]
        u2 = jnp.concatenate([u, u], axis=1)
        mlo_m = (jnp.arange(54)[None, :] < 27).astype(jnp.float32)
        bmv = bmr[...]
        ms_lo = jnp.sum(bmv * u2 * mlo_m, axis=1, keepdims=True) + const
        ms_hi = jnp.sum(bmv * u2 * (1.0 - mlo_m), axis=1, keepdims=True) + const
        w2 = jnp.concatenate([w, w], axis=1)
        dlo = (jnp.arange(2 * _D)[None, :] < _D).astype(jnp.float32)
        pos = jnp.maximum(hs[...] + hp[...], 0.0) * w2
        neg = jnp.maximum(hs[...] + hn[...], 0.0) * w2
        opos[...] = jnp.concatenate(
            [jnp.sum(pos * dlo, axis=1, keepdims=True) + ms_lo,
             jnp.sum(pos * (1.0 - dlo), axis=1, keepdims=True) + ms_hi], axis=1)
        oneg[...] = jnp.concatenate(
            [jnp.sum(neg * dlo, axis=1, keepdims=True) + ms_lo,
             jnp.sum(neg * (1.0 - dlo), axis=1, keepdims=True) + ms_hi], axis=1)

    return pl.pallas_call(
        body,
        grid=(_NB2,),
        in_specs=[
            pl.BlockSpec((_BLK2, 2 * _D), lambda i: (i, 0)),
            pl.BlockSpec((_BLK2, 2 * _D), lambda i: (i + _NB2, 0)),
            pl.BlockSpec((_BLK2, 2 * _D), lambda i: (i + 2 * _NB2, 0)),
            pl.BlockSpec((_BLK2, 54), lambda i: (i, 0)),
            pl.BlockSpec((_D, 27), lambda i: (0, 0)),
            pl.BlockSpec((_D,), lambda i: (0,)),
            pl.BlockSpec((1, _D), lambda i: (0, 0)),
            pl.BlockSpec((1,), lambda i: (0,)),
        ],
        out_specs=[
            pl.BlockSpec((_BLK2, 2), lambda i: (i, 0)),
            pl.BlockSpec((_BLK2, 2), lambda i: (i, 0)),
        ],
        out_shape=[
            jax.ShapeDtypeStruct((_B // 2, 2), jnp.float32),
            jax.ShapeDtypeStruct((_B // 2, 2), jnp.float32),
        ],
    )(h2, h2, h2, bm2, ew, eb, ow, ob)


def kernel(x, src, dst, neg_dst, batch_msg,
           emb0, emb1, emb2, emb3, emb4, emb5, emb6, emb7, emb8, emb9,
           edge_W, edge_b, out_W, out_b):
    xt = x.T
    xcs = [xt[f] for f in range(10)]
    idx_all = jnp.concatenate([src, dst, neg_dst], axis=0)
    embs2 = [_tc_transpose_table(jnp.pad(e.T, ((0, 0), (0, _VP - _V))))
             for e in (emb1, emb2, emb3, emb4, emb5, emb6, emb7, emb8, emb9)]
    h2 = _sc_encode(*xcs, idx_all, emb0.reshape(-1), *embs2)
    bm2 = batch_msg.reshape(_B // 2, 54)
    op2, on2 = _tc_epilogue(h2, bm2, edge_W, edge_b, out_W, out_b)
    return (op2.reshape(_B, 1), on2.reshape(_B, 1))


# speedup vs baseline: 1.4368x; 1.2917x over previous
import functools

import jax
import jax.numpy as jnp
from jax import lax
from jax.experimental import pallas as pl
from jax.experimental.pallas import tpu as pltpu
from jax.experimental.pallas import tpu_sc as plsc

_V = 100000
_D = 64
_B = 16384
_R = 3 * _B
_NC, _NS, _L = 2, 16, 16
_NW = _NC * _NS
_RPW = _R // _NW
_C = 64
_NCH = _RPW // _C

_VP = 102400
_TBLK = 12800


def _tc_transpose_table(et):
    def body(inr, outr):
        outr[:, 0:_D] = jnp.transpose(inr[...])

    return pl.pallas_call(
        body,
        grid=(_VP // _TBLK,),
        in_specs=[pl.BlockSpec((_D, _TBLK), lambda i: (0, i))],
        out_specs=pl.BlockSpec((_TBLK, 2 * _D), lambda i: (i, 0)),
        out_shape=jax.ShapeDtypeStruct((_VP, 2 * _D), jnp.float32),
    )(et)


@functools.partial(
    pl.kernel,
    mesh=plsc.VectorSubcoreMesh(core_axis_name="c", subcore_axis_name="s"),
    out_type=jax.ShapeDtypeStruct((_R // 2, 2 * _D), jnp.float32),
    compiler_params=pltpu.CompilerParams(use_tc_tiling_on_sc=False),
    scratch_types=[
        pltpu.VMEM((_C,), jnp.int32),
        pltpu.VMEM((9, _C), jnp.int32),
        pltpu.VMEM((_C + _L,), jnp.int32),
        pltpu.VMEM((9, _C, 2 * _D), jnp.float32),
        pltpu.VMEM((_C // 2, 2 * _D), jnp.float32),
        pltpu.VMEM((3 * _D,), jnp.float32),
        pltpu.SemaphoreType.DMA,
    ],
)
def _sc_encode(xc0, xc1, xc2, xc3, xc4, xc5, xc6, xc7, xc8, xc9,
               idx_all, e0, e1, e2, e3, e4, e5, e6, e7, e8, e9,
               hout, idx_v, cols, tlist, rbuf, hloc, emb0v, sem):
    embs = (e1, e2, e3, e4, e5, e6, e7, e8, e9)
    xcs = (xc1, xc2, xc3, xc4, xc5, xc6, xc7, xc8, xc9)
    wid = lax.axis_index("s") * _NC + lax.axis_index("c")
    pltpu.sync_copy(e0, emb0v)
    e0sl = [[emb0v[pl.ds(t * _D + dv * _L, _L)] for dv in range(_D // _L)]
            for t in range(3)]
    base_w = wid * _RPW

    def chunk(ci, carry):
        base = base_w + ci * _C
        pltpu.sync_copy(idx_all.at[pl.ds(base, _C)], idx_v)
        xcps = [pltpu.async_copy(xc0.at[idx_v], tlist.at[pl.ds(0, _C)], sem)]
        xcps += [pltpu.async_copy(xcs[f].at[idx_v], cols.at[f], sem)
                 for f in range(9)]
        for cp in xcps:
            cp.wait()
        cps = [pltpu.async_copy(embs[f].at[cols.at[f]], rbuf.at[f], sem)
               for f in range(9)]
        for cp in cps:
            cp.wait()

        def row(r, c2):
            t = tlist[pl.ds(r, _L)][0]
            s0 = jnp.where(t == 0, 1.0, 0.0)
            s1 = jnp.where(t == 1, 1.0, 0.0)
            s2 = jnp.where(t == 2, 1.0, 0.0)
            b0 = lax.broadcast(s0, (_L,))
            b1 = lax.broadcast(s1, (_L,))
            b2 = lax.broadcast(s2, (_L,))
            hrow = r >> 1
            hcol = (r & 1) * _D
            for dv in range(_D // _L):
                sl = pl.ds(dv * _L, _L)
                h0 = (e0sl[0][dv] * b0 + e0sl[1][dv] * b1 + e0sl[2][dv] * b2)
                a = (rbuf[0, r, sl] + rbuf[1, r, sl]
                     + rbuf[2, r, sl] + rbuf[3, r, sl])
                b = rbuf[4, r, sl]
                c = (rbuf[5, r, sl] + rbuf[6, r, sl]
                     + rbuf[7, r, sl] + rbuf[8, r, sl])
                hloc[hrow, pl.ds(hcol + dv * _L, _L)] = (
                    h0 + a * b0 + b * b1 + c * b2)
            return c2

        lax.fori_loop(0, _C, row, 0)
        pltpu.sync_copy(hloc, hout.at[pl.ds(base // 2, _C // 2)])
        return carry

    lax.fori_loop(0, _NCH, chunk, 0)


_BLK2 = 1024
_NB2 = (_B // 2) // _BLK2


def _tc_epilogue(h2, bm2, ew, eb, ow, ob):
    def body(hs, hp, hn, bmr, ewr, ebr, owr, obr, opos, oneg):
        w = owr[...]
        u = jnp.sum(w.T * ewr[...], axis=0, keepdims=True)
        const = jnp.sum(ebr[...] * w[0]) + obr[...][0]
        u2 = jnp.concatenate([u, u], axis=1)
        mlo_m = (jnp.arange(54)[None, :] < 27).astype(jnp.float32)
        bmv = bmr[...]
        ms_lo = jnp.sum(bmv * u2 * mlo_m, axis=1, keepdims=True) + const
        ms_hi = jnp.sum(bmv * u2 * (1.0 - mlo_m), axis=1, keepdims=True) + const
        w2 = jnp.concatenate([w, w], axis=1)
        dlo = (jnp.arange(2 * _D)[None, :] < _D).astype(jnp.float32)
        pos = jnp.maximum(hs[...] + hp[...], 0.0) * w2
        neg = jnp.maximum(hs[...] + hn[...], 0.0) * w2
        opos[...] = jnp.concatenate(
            [jnp.sum(pos * dlo, axis=1, keepdims=True) + ms_lo,
             jnp.sum(pos * (1.0 - dlo), axis=1, keepdims=True) + ms_hi], axis=1)
        oneg[...] = jnp.concatenate(
            [jnp.sum(neg * dlo, axis=1, keepdims=True) + ms_lo,
             jnp.sum(neg * (1.0 - dlo), axis=1, keepdims=True) + ms_hi], axis=1)

    return pl.pallas_call(
        body,
        grid=(_NB2,),
        in_specs=[
            pl.BlockSpec((_BLK2, 2 * _D), lambda i: (i, 0)),
            pl.BlockSpec((_BLK2, 2 * _D), lambda i: (i + _NB2, 0)),
            pl.BlockSpec((_BLK2, 2 * _D), lambda i: (i + 2 * _NB2, 0)),
            pl.BlockSpec((_BLK2, 54), lambda i: (i, 0)),
            pl.BlockSpec((_D, 27), lambda i: (0, 0)),
            pl.BlockSpec((_D,), lambda i: (0,)),
            pl.BlockSpec((1, _D), lambda i: (0, 0)),
            pl.BlockSpec((1,), lambda i: (0,)),
        ],
        out_specs=[
            pl.BlockSpec((_BLK2, 2), lambda i: (i, 0)),
            pl.BlockSpec((_BLK2, 2), lambda i: (i, 0)),
        ],
        out_shape=[
            jax.ShapeDtypeStruct((_B // 2, 2), jnp.float32),
            jax.ShapeDtypeStruct((_B // 2, 2), jnp.float32),
        ],
    )(h2, h2, h2, bm2, ew, eb, ow, ob)


def kernel(x, src, dst, neg_dst, batch_msg,
           emb0, emb1, emb2, emb3, emb4, emb5, emb6, emb7, emb8, emb9,
           edge_W, edge_b, out_W, out_b):
    xt = x.T
    xcs = [xt[f] for f in range(10)]
    idx_all = jnp.concatenate([src, dst, neg_dst], axis=0)
    embs2 = [_tc_transpose_table(e.T)
             for e in (emb1, emb2, emb3, emb4, emb5, emb6, emb7, emb8, emb9)]
    h2 = _sc_encode(*xcs, idx_all, emb0.reshape(-1), *embs2)
    bm2 = batch_msg.reshape(_B // 2, 54)
    op2, on2 = _tc_epilogue(h2, bm2, edge_W, edge_b, out_W, out_b)
    return (op2.reshape(_B, 1), on2.reshape(_B, 1))
